# baseline (speedup 1.0000x reference)
import functools

import jax
import jax.numpy as jnp
from jax import lax
from jax.experimental import pallas as pl
from jax.experimental.pallas import tpu as pltpu
from jax.experimental.pallas import tpu_sc as plsc

N, D, H, E = 2048, 1024, 2048, 8
D2 = D // 2
TILE = 256
MAX_TILES = 24
MAX_ROWS = MAX_TILES * TILE
EMAP_PAD = 32
GW = 128

_HI = jax.lax.Precision.HIGHEST


def _router_body(x_ref, gate_ref, r1_ref, r2_ref, g1_ref, g2_ref, emap_ref,
                 m_ref, incl_ref):
    logits = jnp.dot(x_ref[...], gate_ref[...],
                     preferred_element_type=jnp.float32)
    idx8 = lax.broadcasted_iota(jnp.int32, (N, E), 1)
    m1 = jnp.max(logits, axis=1, keepdims=True)
    e1 = jnp.min(jnp.where(logits == m1, idx8, E), axis=1, keepdims=True)
    oh1 = idx8 == e1
    l2 = jnp.where(oh1, -jnp.inf, logits)
    m2 = jnp.max(l2, axis=1, keepdims=True)
    e2 = jnp.min(jnp.where(l2 == m2, idx8, E), axis=1, keepdims=True)
    oh2 = idx8 == e2
    u = jnp.exp(m2 - m1)
    g1_ref[...] = jnp.broadcast_to(1.0 / (1.0 + u), (N, GW))
    g2_ref[...] = jnp.broadcast_to(u / (1.0 + u), (N, GW))

    mask = (oh1 | oh2).astype(jnp.float32)
    m_ref[...] = mask
    rr = lax.broadcasted_iota(jnp.int32, (TILE, TILE), 0)
    cc = lax.broadcasted_iota(jnp.int32, (TILE, TILE), 1)
    ltri = (rr >= cc).astype(jnp.float32)

    def body(i, base):
        chunk = m_ref[pl.ds(i * TILE, TILE), :]
        incl = jnp.dot(ltri, chunk, preferred_element_type=jnp.float32,
                       precision=_HI) + base
        incl_ref[pl.ds(i * TILE, TILE), :] = incl
        return incl[TILE - 1:TILE, :]

    counts = lax.fori_loop(0, N // TILE, body, jnp.zeros((1, E), jnp.float32))

    pc = jnp.floor((counts + (TILE - 1)) / TILE) * TILE
    r8 = lax.broadcasted_iota(jnp.int32, (E, E), 0)
    c8 = lax.broadcasted_iota(jnp.int32, (E, E), 1)
    ustrict = (r8 < c8).astype(jnp.float32)
    base_e = jnp.dot(pc, ustrict, preferred_element_type=jnp.float32,
                     precision=_HI)

    excl = incl_ref[...] - mask
    dest = excl + base_e
    r1_ref[...] = jnp.sum(jnp.where(oh1, dest, 0.0), axis=1,
                          keepdims=True).astype(jnp.int32)
    r2_ref[...] = jnp.sum(jnp.where(oh2, dest, 0.0), axis=1,
                          keepdims=True).astype(jnp.int32)

    starts = base_e / TILE
    tt = lax.broadcasted_iota(jnp.int32, (EMAP_PAD, E), 0).astype(jnp.float32)
    ge = (tt >= starts).astype(jnp.float32)
    emap_ref[...] = (jnp.sum(ge, axis=1, keepdims=True) - 1.0).astype(jnp.int32)


def _router(x2d, gate):
    return pl.pallas_call(
        _router_body,
        in_specs=[
            pl.BlockSpec((N, D), lambda: (0, 0)),
            pl.BlockSpec((D, E), lambda: (0, 0)),
        ],
        out_specs=[
            pl.BlockSpec((N, 1), lambda: (0, 0)),
            pl.BlockSpec((N, 1), lambda: (0, 0)),
            pl.BlockSpec((N, GW), lambda: (0, 0)),
            pl.BlockSpec((N, GW), lambda: (0, 0)),
            pl.BlockSpec((EMAP_PAD, 1), lambda: (0, 0)),
        ],
        out_shape=[
            jax.ShapeDtypeStruct((N, 1), jnp.int32),
            jax.ShapeDtypeStruct((N, 1), jnp.int32),
            jax.ShapeDtypeStruct((N, GW), jnp.float32),
            jax.ShapeDtypeStruct((N, GW), jnp.float32),
            jax.ShapeDtypeStruct((EMAP_PAD, 1), jnp.int32),
        ],
        scratch_shapes=[
            pltpu.VMEM((N, E), jnp.float32),
            pltpu.VMEM((N, E), jnp.float32),
        ],
    )(x2d, gate)


_NC = 2
_NW = 32
TOK_W = N // _NW
CHUNK = 32


def _sc_scatter_body(x_hbm, r1_hbm, r2_hbm, g1_hbm, g2_hbm, xs_hbm, gs_hbm,
                     i1v, i2v, xbuf, g1buf, g2buf, sem):
    wid = lax.axis_index("s") * _NC + lax.axis_index("c")
    base = wid * TOK_W
    pltpu.sync_copy(x_hbm.at[pl.ds(base, TOK_W)], xbuf)
    pltpu.sync_copy(r1_hbm.at[pl.ds(base, TOK_W)], i1v)
    pltpu.sync_copy(r2_hbm.at[pl.ds(base, TOK_W)], i2v)
    pltpu.sync_copy(g1_hbm.at[pl.ds(base, TOK_W)], g1buf)
    pltpu.sync_copy(g2_hbm.at[pl.ds(base, TOK_W)], g2buf)
    pltpu.async_copy(xbuf, xs_hbm.at[i1v], sem).wait()
    pltpu.async_copy(xbuf, xs_hbm.at[i2v], sem).wait()
    pltpu.async_copy(g1buf, gs_hbm.at[i1v], sem).wait()
    pltpu.async_copy(g2buf, gs_hbm.at[i2v], sem).wait()


def _sc_combine_body(y_hbm, r1_hbm, r2_hbm, out_hbm, i1v, i2v, buf1, buf2,
                     sem1, sem2):
    wid = lax.axis_index("s") * _NC + lax.axis_index("c")
    for c in range(TOK_W // CHUNK):
        base = wid * TOK_W + c * CHUNK
        pltpu.sync_copy(r1_hbm.at[pl.ds(base, CHUNK)], i1v)
        pltpu.sync_copy(r2_hbm.at[pl.ds(base, CHUNK)], i2v)
        cp1 = pltpu.async_copy(y_hbm.at[i1v], buf1, sem1)
        cp2 = pltpu.async_copy(y_hbm.at[i2v], buf2, sem2)
        cp1.wait()
        cp2.wait()

        def tok(i, _):
            for j in range(D // 16):
                sl = pl.ds(j * 16, 16)
                buf1[i, sl] = buf1[i, sl] + buf2[i, sl]
            return 0

        lax.fori_loop(0, CHUNK, tok, 0)
        pltpu.sync_copy(buf1, out_hbm.at[pl.ds(base, CHUNK)])


@functools.cache
def _sc_kernels():
    mesh = plsc.VectorSubcoreMesh(core_axis_name="c", subcore_axis_name="s")
    scatter = pl.kernel(
        _sc_scatter_body,
        mesh=mesh,
        out_type=[
            jax.ShapeDtypeStruct((MAX_ROWS, D2), jnp.int32),
            jax.ShapeDtypeStruct((MAX_ROWS, GW), jnp.float32),
        ],
        scratch_types=[
            pltpu.VMEM((TOK_W,), jnp.int32),
            pltpu.VMEM((TOK_W,), jnp.int32),
            pltpu.VMEM((TOK_W, D2), jnp.int32),
            pltpu.VMEM((TOK_W, GW), jnp.float32),
            pltpu.VMEM((TOK_W, GW), jnp.float32),
            pltpu.SemaphoreType.DMA,
        ],
    )
    combine = pl.kernel(
        _sc_combine_body,
        mesh=mesh,
        out_type=jax.ShapeDtypeStruct((N, D), jnp.float32),
        scratch_types=[
            pltpu.VMEM((CHUNK,), jnp.int32),
            pltpu.VMEM((CHUNK,), jnp.int32),
            pltpu.VMEM((CHUNK, D), jnp.float32),
            pltpu.VMEM((CHUNK, D), jnp.float32),
            pltpu.SemaphoreType.DMA,
            pltpu.SemaphoreType.DMA,
        ],
    )
    return scatter, combine


def _gemm_body(emap_ref, xs_ref, w1_ref, b1_ref, w2_ref, b2_ref, gs_ref,
               y_ref):
    del emap_ref
    xb = xs_ref[...]
    w1b = w1_ref[0].astype(jnp.bfloat16)
    w2b = w2_ref[0].astype(jnp.bfloat16)
    h = jnp.maximum(
        jnp.dot(xb, w1b, preferred_element_type=jnp.float32) + b1_ref[0],
        0.0,
    ).astype(jnp.bfloat16)
    y = jnp.dot(h, w2b, preferred_element_type=jnp.float32) + b2_ref[0]
    y_ref[...] = y * gs_ref[:, 0:1]


def _gemm(emap, xs_bf, w1, b1, w2, b2, gs):
    grid_spec = pltpu.PrefetchScalarGridSpec(
        num_scalar_prefetch=1,
        grid=(MAX_TILES,),
        in_specs=[
            pl.BlockSpec((TILE, D), lambda t, emap: (t, 0)),
            pl.BlockSpec((1, D, H), lambda t, emap: (emap[t], 0, 0)),
            pl.BlockSpec((1, 1, H), lambda t, emap: (emap[t], 0, 0)),
            pl.BlockSpec((1, H, D), lambda t, emap: (emap[t], 0, 0)),
            pl.BlockSpec((1, 1, D), lambda t, emap: (emap[t], 0, 0)),
            pl.BlockSpec((TILE, GW), lambda t, emap: (t, 0)),
        ],
        out_specs=pl.BlockSpec((TILE, D), lambda t, emap: (t, 0)),
    )
    return pl.pallas_call(
        _gemm_body,
        grid_spec=grid_spec,
        out_shape=jax.ShapeDtypeStruct((MAX_ROWS, D), jnp.float32),
        compiler_params=pltpu.CompilerParams(
            dimension_semantics=("arbitrary",),
        ),
    )(emap, xs_bf, w1, b1.reshape(E, 1, H), w2, b2.reshape(E, 1, D), gs)


def kernel(x, gate, w1, b1, w2, b2):
    x2d = x.reshape(N, D)
    r1, r2, g1, g2, emap = _router(x2d, gate)
    r1f, r2f = r1.reshape(N), r2.reshape(N)
    xi = lax.bitcast_convert_type(
        x2d.astype(jnp.bfloat16).reshape(N, D2, 2), jnp.int32)
    sc_scatter, sc_combine = _sc_kernels()
    xs_i32, gs = sc_scatter(xi, r1f, r2f, g1, g2)
    xs_bf = lax.bitcast_convert_type(xs_i32, jnp.bfloat16).reshape(MAX_ROWS, D)
    y = _gemm(emap.reshape(EMAP_PAD), xs_bf, w1, b1, w2, b2, gs)
    out = sc_combine(y, r1f, r2f)
    return out.reshape(1, N, D)


# speedup vs baseline: 2.0196x; 2.0196x over previous
import functools

import jax
import jax.numpy as jnp
from jax import lax
from jax.experimental import pallas as pl
from jax.experimental.pallas import tpu as pltpu
from jax.experimental.pallas import tpu_sc as plsc

N, D, H, E = 2048, 1024, 2048, 8
TILE = 256
MAX_TILES = 24
MAX_ROWS = MAX_TILES * TILE
EMAP_PAD = 32

_HI = jax.lax.Precision.HIGHEST


def _router_body(x_ref, gate_ref, r1_ref, r2_ref, g1_ref, g2_ref, emap_ref,
                 m_ref, incl_ref):
    logits = jnp.dot(x_ref[...], gate_ref[...],
                     preferred_element_type=jnp.float32)
    idx8 = lax.broadcasted_iota(jnp.int32, (N, E), 1)
    m1 = jnp.max(logits, axis=1, keepdims=True)
    e1 = jnp.min(jnp.where(logits == m1, idx8, E), axis=1, keepdims=True)
    oh1 = idx8 == e1
    l2 = jnp.where(oh1, -jnp.inf, logits)
    m2 = jnp.max(l2, axis=1, keepdims=True)
    e2 = jnp.min(jnp.where(l2 == m2, idx8, E), axis=1, keepdims=True)
    oh2 = idx8 == e2
    u = jnp.exp(m2 - m1)
    g1_ref[...] = 1.0 / (1.0 + u)
    g2_ref[...] = u / (1.0 + u)

    mask = (oh1 | oh2).astype(jnp.float32)
    m_ref[...] = mask
    rr = lax.broadcasted_iota(jnp.int32, (TILE, TILE), 0)
    cc = lax.broadcasted_iota(jnp.int32, (TILE, TILE), 1)
    ltri = (rr >= cc).astype(jnp.float32)

    def body(i, base):
        chunk = m_ref[pl.ds(i * TILE, TILE), :]
        incl = jnp.dot(ltri, chunk, preferred_element_type=jnp.float32,
                       precision=_HI) + base
        incl_ref[pl.ds(i * TILE, TILE), :] = incl
        return incl[TILE - 1:TILE, :]

    counts = lax.fori_loop(0, N // TILE, body, jnp.zeros((1, E), jnp.float32))

    pc = jnp.floor((counts + (TILE - 1)) / TILE) * TILE
    r8 = lax.broadcasted_iota(jnp.int32, (E, E), 0)
    c8 = lax.broadcasted_iota(jnp.int32, (E, E), 1)
    ustrict = (r8 < c8).astype(jnp.float32)
    base_e = jnp.dot(pc, ustrict, preferred_element_type=jnp.float32,
                     precision=_HI)

    excl = incl_ref[...] - mask
    dest = excl + base_e
    r1_ref[...] = jnp.sum(jnp.where(oh1, dest, 0.0), axis=1,
                          keepdims=True).astype(jnp.int32)
    r2_ref[...] = jnp.sum(jnp.where(oh2, dest, 0.0), axis=1,
                          keepdims=True).astype(jnp.int32)

    starts = base_e / TILE
    tt = lax.broadcasted_iota(jnp.int32, (EMAP_PAD, E), 0).astype(jnp.float32)
    ge = (tt >= starts).astype(jnp.float32)
    emap_ref[...] = (jnp.sum(ge, axis=1, keepdims=True) - 1.0).astype(jnp.int32)


def _router(x2d, gate):
    return pl.pallas_call(
        _router_body,
        in_specs=[
            pl.BlockSpec((N, D), lambda: (0, 0)),
            pl.BlockSpec((D, E), lambda: (0, 0)),
        ],
        out_specs=[
            pl.BlockSpec((N, 1), lambda: (0, 0)),
            pl.BlockSpec((N, 1), lambda: (0, 0)),
            pl.BlockSpec((N, 1), lambda: (0, 0)),
            pl.BlockSpec((N, 1), lambda: (0, 0)),
            pl.BlockSpec((EMAP_PAD, 1), lambda: (0, 0)),
        ],
        out_shape=[
            jax.ShapeDtypeStruct((N, 1), jnp.int32),
            jax.ShapeDtypeStruct((N, 1), jnp.int32),
            jax.ShapeDtypeStruct((N, 1), jnp.float32),
            jax.ShapeDtypeStruct((N, 1), jnp.float32),
            jax.ShapeDtypeStruct((EMAP_PAD, 1), jnp.int32),
        ],
        scratch_shapes=[
            pltpu.VMEM((N, E), jnp.float32),
            pltpu.VMEM((N, E), jnp.float32),
        ],
    )(x2d, gate)


_NC = 2
_NW = 32
TOK_W = N // _NW
CHUNK = 32


def _sc_scatter_body(x_hbm, r1_hbm, r2_hbm, xs_hbm, i1v, i2v, xbuf, sem):
    wid = lax.axis_index("s") * _NC + lax.axis_index("c")
    base = wid * TOK_W
    pltpu.sync_copy(x_hbm.at[pl.ds(base, TOK_W)], xbuf)
    pltpu.sync_copy(r1_hbm.at[pl.ds(base, TOK_W)], i1v)
    pltpu.sync_copy(r2_hbm.at[pl.ds(base, TOK_W)], i2v)
    pltpu.async_copy(xbuf, xs_hbm.at[i1v], sem).wait()
    pltpu.async_copy(xbuf, xs_hbm.at[i2v], sem).wait()


def _lane_bcast(v16, lane):
    idx = jnp.full((16, 1), lane, jnp.int32)
    dn = lax.GatherDimensionNumbers(
        offset_dims=(), collapsed_slice_dims=(0,), start_index_map=(0,))
    return lax.gather(v16, idx, dn, (1,),
                      mode=lax.GatherScatterMode.PROMISE_IN_BOUNDS)


def _sc_combine_body(y_hbm, r1_hbm, r2_hbm, g1_hbm, g2_hbm, out_hbm,
                     i1v, i2v, g1v, g2v, buf1, buf2, sem1, sem2):
    wid = lax.axis_index("s") * _NC + lax.axis_index("c")
    for c in range(TOK_W // CHUNK):
        base = wid * TOK_W + c * CHUNK
        pltpu.sync_copy(r1_hbm.at[pl.ds(base, CHUNK)], i1v)
        pltpu.sync_copy(r2_hbm.at[pl.ds(base, CHUNK)], i2v)
        pltpu.sync_copy(g1_hbm.at[pl.ds(base, CHUNK)], g1v)
        pltpu.sync_copy(g2_hbm.at[pl.ds(base, CHUNK)], g2v)
        cp1 = pltpu.async_copy(y_hbm.at[i1v], buf1, sem1)
        cp2 = pltpu.async_copy(y_hbm.at[i2v], buf2, sem2)
        cp1.wait()
        cp2.wait()
        for grp in range(CHUNK // 16):
            ga16 = g1v[pl.ds(grp * 16, 16)]
            gb16 = g2v[pl.ds(grp * 16, 16)]

            def tok(i2, _):
                ga = _lane_bcast(ga16, i2)
                gb = _lane_bcast(gb16, i2)
                row = grp * 16 + i2
                for j in range(D // 16):
                    sl = pl.ds(j * 16, 16)
                    buf1[row, sl] = buf1[row, sl] * ga + buf2[row, sl] * gb
                return 0

            lax.fori_loop(0, 16, tok, 0)
        pltpu.sync_copy(buf1, out_hbm.at[pl.ds(base, CHUNK)])


@functools.cache
def _sc_kernels():
    mesh = plsc.VectorSubcoreMesh(core_axis_name="c", subcore_axis_name="s")
    scatter = pl.kernel(
        _sc_scatter_body,
        mesh=mesh,
        out_type=jax.ShapeDtypeStruct((MAX_ROWS, D), jnp.float32),
        scratch_types=[
            pltpu.VMEM((TOK_W,), jnp.int32),
            pltpu.VMEM((TOK_W,), jnp.int32),
            pltpu.VMEM((TOK_W, D), jnp.float32),
            pltpu.SemaphoreType.DMA,
        ],
    )
    combine = pl.kernel(
        _sc_combine_body,
        mesh=mesh,
        out_type=jax.ShapeDtypeStruct((N, D), jnp.float32),
        scratch_types=[
            pltpu.VMEM((CHUNK,), jnp.int32),
            pltpu.VMEM((CHUNK,), jnp.int32),
            pltpu.VMEM((CHUNK,), jnp.float32),
            pltpu.VMEM((CHUNK,), jnp.float32),
            pltpu.VMEM((CHUNK, D), jnp.float32),
            pltpu.VMEM((CHUNK, D), jnp.float32),
            pltpu.SemaphoreType.DMA,
            pltpu.SemaphoreType.DMA,
        ],
    )
    return scatter, combine


def _gemm_body(emap_ref, xs_ref, w1_ref, b1_ref, w2_ref, b2_ref, y_ref):
    del emap_ref
    xb = xs_ref[...].astype(jnp.bfloat16)
    w1b = w1_ref[0].astype(jnp.bfloat16)
    w2b = w2_ref[0].astype(jnp.bfloat16)
    h = jnp.maximum(
        jnp.dot(xb, w1b, preferred_element_type=jnp.float32) + b1_ref[0],
        0.0,
    ).astype(jnp.bfloat16)
    y_ref[...] = (
        jnp.dot(h, w2b, preferred_element_type=jnp.float32) + b2_ref[0]
    )


def _gemm(emap, xs, w1, b1, w2, b2):
    grid_spec = pltpu.PrefetchScalarGridSpec(
        num_scalar_prefetch=1,
        grid=(MAX_TILES,),
        in_specs=[
            pl.BlockSpec((TILE, D), lambda t, emap: (t, 0)),
            pl.BlockSpec((1, D, H), lambda t, emap: (emap[t], 0, 0)),
            pl.BlockSpec((1, 1, H), lambda t, emap: (emap[t], 0, 0)),
            pl.BlockSpec((1, H, D), lambda t, emap: (emap[t], 0, 0)),
            pl.BlockSpec((1, 1, D), lambda t, emap: (emap[t], 0, 0)),
        ],
        out_specs=pl.BlockSpec((TILE, D), lambda t, emap: (t, 0)),
    )
    return pl.pallas_call(
        _gemm_body,
        grid_spec=grid_spec,
        out_shape=jax.ShapeDtypeStruct((MAX_ROWS, D), jnp.float32),
        compiler_params=pltpu.CompilerParams(
            dimension_semantics=("arbitrary",),
        ),
    )(emap, xs, w1, b1.reshape(E, 1, H), w2, b2.reshape(E, 1, D))


def kernel(x, gate, w1, b1, w2, b2):
    x2d = x.reshape(N, D)
    r1, r2, g1, g2, emap = _router(x2d, gate)
    r1f, r2f = r1.reshape(N), r2.reshape(N)
    sc_scatter, sc_combine = _sc_kernels()
    xs = sc_scatter(x2d, r1f, r2f)
    y = _gemm(emap.reshape(EMAP_PAD), xs, w1, b1, w2, b2)
    out = sc_combine(y, r1f, r2f, g1.reshape(N), g2.reshape(N))
    return out.reshape(1, N, D)


# speedup vs baseline: 2.0663x; 1.0231x over previous
import functools

import jax
import jax.numpy as jnp
from jax import lax
from jax.experimental import pallas as pl
from jax.experimental.pallas import tpu as pltpu
from jax.experimental.pallas import tpu_sc as plsc

N, D, H, E = 2048, 1024, 2048, 8
TILE = 256
MAX_TILES = 24
MAX_ROWS = MAX_TILES * TILE
EMAP_PAD = 32

_HI = jax.lax.Precision.HIGHEST


def _router_body(x_ref, gate_ref, r1_ref, r2_ref, g1_ref, g2_ref, emap_ref,
                 m_ref, incl_ref):
    logits = jnp.dot(x_ref[...], gate_ref[...],
                     preferred_element_type=jnp.float32)
    idx8 = lax.broadcasted_iota(jnp.int32, (N, E), 1)
    m1 = jnp.max(logits, axis=1, keepdims=True)
    e1 = jnp.min(jnp.where(logits == m1, idx8, E), axis=1, keepdims=True)
    oh1 = idx8 == e1
    l2 = jnp.where(oh1, -jnp.inf, logits)
    m2 = jnp.max(l2, axis=1, keepdims=True)
    e2 = jnp.min(jnp.where(l2 == m2, idx8, E), axis=1, keepdims=True)
    oh2 = idx8 == e2
    u = jnp.exp(m2 - m1)
    g1_ref[...] = 1.0 / (1.0 + u)
    g2_ref[...] = u / (1.0 + u)

    mask = (oh1 | oh2).astype(jnp.float32)
    m_ref[...] = mask
    rr = lax.broadcasted_iota(jnp.int32, (TILE, TILE), 0)
    cc = lax.broadcasted_iota(jnp.int32, (TILE, TILE), 1)
    ltri = (rr >= cc).astype(jnp.float32)

    def body(i, base):
        chunk = m_ref[pl.ds(i * TILE, TILE), :]
        incl = jnp.dot(ltri, chunk, preferred_element_type=jnp.float32,
                       precision=_HI) + base
        incl_ref[pl.ds(i * TILE, TILE), :] = incl
        return incl[TILE - 1:TILE, :]

    counts = lax.fori_loop(0, N // TILE, body, jnp.zeros((1, E), jnp.float32))

    pc = jnp.floor((counts + (TILE - 1)) / TILE) * TILE
    r8 = lax.broadcasted_iota(jnp.int32, (E, E), 0)
    c8 = lax.broadcasted_iota(jnp.int32, (E, E), 1)
    ustrict = (r8 < c8).astype(jnp.float32)
    base_e = jnp.dot(pc, ustrict, preferred_element_type=jnp.float32,
                     precision=_HI)

    excl = incl_ref[...] - mask
    dest = excl + base_e
    r1_ref[...] = jnp.sum(jnp.where(oh1, dest, 0.0), axis=1,
                          keepdims=True).astype(jnp.int32)
    r2_ref[...] = jnp.sum(jnp.where(oh2, dest, 0.0), axis=1,
                          keepdims=True).astype(jnp.int32)

    starts = base_e / TILE
    tt = lax.broadcasted_iota(jnp.int32, (EMAP_PAD, E), 0).astype(jnp.float32)
    ge = (tt >= starts).astype(jnp.float32)
    emap_ref[...] = (jnp.sum(ge, axis=1, keepdims=True) - 1.0).astype(jnp.int32)


def _router(x2d, gate):
    return pl.pallas_call(
        _router_body,
        in_specs=[
            pl.BlockSpec((N, D), lambda: (0, 0)),
            pl.BlockSpec((D, E), lambda: (0, 0)),
        ],
        out_specs=[
            pl.BlockSpec((N, 1), lambda: (0, 0)),
            pl.BlockSpec((N, 1), lambda: (0, 0)),
            pl.BlockSpec((N, 1), lambda: (0, 0)),
            pl.BlockSpec((N, 1), lambda: (0, 0)),
            pl.BlockSpec((EMAP_PAD, 1), lambda: (0, 0)),
        ],
        out_shape=[
            jax.ShapeDtypeStruct((N, 1), jnp.int32),
            jax.ShapeDtypeStruct((N, 1), jnp.int32),
            jax.ShapeDtypeStruct((N, 1), jnp.float32),
            jax.ShapeDtypeStruct((N, 1), jnp.float32),
            jax.ShapeDtypeStruct((EMAP_PAD, 1), jnp.int32),
        ],
        scratch_shapes=[
            pltpu.VMEM((N, E), jnp.float32),
            pltpu.VMEM((N, E), jnp.float32),
        ],
    )(x2d, gate)


_NC = 2
_NW = 32
TOK_W = N // _NW
CHUNK = 16


def _sc_scatter_body(x_hbm, r1_hbm, r2_hbm, xs_hbm, i1v, i2v, xbuf, sem, sem2):
    wid = lax.axis_index("s") * _NC + lax.axis_index("c")
    base = wid * TOK_W
    pltpu.sync_copy(r1_hbm.at[pl.ds(base, TOK_W)], i1v)
    pltpu.sync_copy(r2_hbm.at[pl.ds(base, TOK_W)], i2v)
    pltpu.sync_copy(x_hbm.at[pl.ds(base, TOK_W)], xbuf)
    d1 = pltpu.async_copy(xbuf, xs_hbm.at[i1v], sem)
    d2 = pltpu.async_copy(xbuf, xs_hbm.at[i2v], sem2)
    d1.wait()
    d2.wait()


def _lane_bcast(v16, lane):
    idx = jnp.full((16, 1), lane, jnp.int32)
    dn = lax.GatherDimensionNumbers(
        offset_dims=(), collapsed_slice_dims=(0,), start_index_map=(0,))
    return lax.gather(v16, idx, dn, (1,),
                      mode=lax.GatherScatterMode.PROMISE_IN_BOUNDS)


def _sc_combine_body(y_hbm, r1_hbm, r2_hbm, g1_hbm, g2_hbm, out_hbm,
                     i1m, i2m, g1v, g2v, buf1, buf2, obuf,
                     s1a, s1b, s2a, s2b, soa, sob):
    wid = lax.axis_index("s") * _NC + lax.axis_index("c")
    base0 = wid * TOK_W
    nch = TOK_W // CHUNK
    for c in range(nch):
        pltpu.sync_copy(r1_hbm.at[pl.ds(base0 + c * CHUNK, CHUNK)], i1m.at[c])
        pltpu.sync_copy(r2_hbm.at[pl.ds(base0 + c * CHUNK, CHUNK)], i2m.at[c])
    pltpu.sync_copy(g1_hbm.at[pl.ds(base0, TOK_W)], g1v)
    pltpu.sync_copy(g2_hbm.at[pl.ds(base0, TOK_W)], g2v)
    s1 = (s1a, s1b)
    s2 = (s2a, s2b)
    so = (soa, sob)
    gcp = [None] * nch
    ocp = [None] * nch

    def issue(c):
        b = c & 1
        gcp[c] = (
            pltpu.async_copy(y_hbm.at[i1m.at[c]], buf1.at[b], s1[b]),
            pltpu.async_copy(y_hbm.at[i2m.at[c]], buf2.at[b], s2[b]),
        )

    def process(c):
        b = c & 1
        if c >= 2:
            ocp[c - 2].wait()
        d1, d2 = gcp[c]
        d1.wait()
        d2.wait()
        ga16 = g1v[pl.ds(c * CHUNK, CHUNK)]
        gb16 = g2v[pl.ds(c * CHUNK, CHUNK)]

        def tok(i2, _):
            ga = _lane_bcast(ga16, i2)
            gb = _lane_bcast(gb16, i2)
            for j in range(D // 16):
                sl = pl.ds(j * 16, 16)
                obuf[b, i2, sl] = buf1[b, i2, sl] * ga + buf2[b, i2, sl] * gb
            return 0

        lax.fori_loop(0, CHUNK, tok, 0)
        ocp[c] = pltpu.async_copy(
            obuf.at[b], out_hbm.at[pl.ds(base0 + c * CHUNK, CHUNK)], so[b])

    issue(0)
    issue(1)
    for c in range(nch):
        process(c)
        if c + 2 < nch:
            issue(c + 2)
    ocp[nch - 2].wait()
    ocp[nch - 1].wait()


@functools.cache
def _sc_kernels():
    mesh = plsc.VectorSubcoreMesh(core_axis_name="c", subcore_axis_name="s")
    scatter = pl.kernel(
        _sc_scatter_body,
        mesh=mesh,
        out_type=jax.ShapeDtypeStruct((MAX_ROWS, D), jnp.float32),
        scratch_types=[
            pltpu.VMEM((TOK_W,), jnp.int32),
            pltpu.VMEM((TOK_W,), jnp.int32),
            pltpu.VMEM((TOK_W, D), jnp.float32),
            pltpu.SemaphoreType.DMA,
            pltpu.SemaphoreType.DMA,
        ],
    )
    combine = pl.kernel(
        _sc_combine_body,
        mesh=mesh,
        out_type=jax.ShapeDtypeStruct((N, D), jnp.float32),
        scratch_types=[
            pltpu.VMEM((TOK_W // CHUNK, CHUNK), jnp.int32),
            pltpu.VMEM((TOK_W // CHUNK, CHUNK), jnp.int32),
            pltpu.VMEM((TOK_W,), jnp.float32),
            pltpu.VMEM((TOK_W,), jnp.float32),
            pltpu.VMEM((2, CHUNK, D), jnp.float32),
            pltpu.VMEM((2, CHUNK, D), jnp.float32),
            pltpu.VMEM((2, CHUNK, D), jnp.float32),
            pltpu.SemaphoreType.DMA,
            pltpu.SemaphoreType.DMA,
            pltpu.SemaphoreType.DMA,
            pltpu.SemaphoreType.DMA,
            pltpu.SemaphoreType.DMA,
            pltpu.SemaphoreType.DMA,
        ],
    )
    return scatter, combine


def _gemm_body(emap_ref, xs_ref, w1_ref, b1_ref, w2_ref, b2_ref, y_ref):
    del emap_ref
    xb = xs_ref[...].astype(jnp.bfloat16)
    w1b = w1_ref[0].astype(jnp.bfloat16)
    w2b = w2_ref[0].astype(jnp.bfloat16)
    h = jnp.maximum(
        jnp.dot(xb, w1b, preferred_element_type=jnp.float32) + b1_ref[0],
        0.0,
    ).astype(jnp.bfloat16)
    y_ref[...] = (
        jnp.dot(h, w2b, preferred_element_type=jnp.float32) + b2_ref[0]
    )


def _gemm(emap, xs, w1, b1, w2, b2):
    grid_spec = pltpu.PrefetchScalarGridSpec(
        num_scalar_prefetch=1,
        grid=(MAX_TILES,),
        in_specs=[
            pl.BlockSpec((TILE, D), lambda t, emap: (t, 0)),
            pl.BlockSpec((1, D, H), lambda t, emap: (emap[t], 0, 0)),
            pl.BlockSpec((1, 1, H), lambda t, emap: (emap[t], 0, 0)),
            pl.BlockSpec((1, H, D), lambda t, emap: (emap[t], 0, 0)),
            pl.BlockSpec((1, 1, D), lambda t, emap: (emap[t], 0, 0)),
        ],
        out_specs=pl.BlockSpec((TILE, D), lambda t, emap: (t, 0)),
    )
    return pl.pallas_call(
        _gemm_body,
        grid_spec=grid_spec,
        out_shape=jax.ShapeDtypeStruct((MAX_ROWS, D), jnp.float32),
        compiler_params=pltpu.CompilerParams(
            dimension_semantics=("arbitrary",),
        ),
    )(emap, xs, w1, b1.reshape(E, 1, H), w2, b2.reshape(E, 1, D))


def kernel(x, gate, w1, b1, w2, b2):
    x2d = x.reshape(N, D)
    r1, r2, g1, g2, emap = _router(x2d, gate)
    r1f, r2f = r1.reshape(N), r2.reshape(N)
    sc_scatter, sc_combine = _sc_kernels()
    xs = sc_scatter(x2d, r1f, r2f)
    y = _gemm(emap.reshape(EMAP_PAD), xs, w1, b1, w2, b2)
    out = sc_combine(y, r1f, r2f, g1.reshape(N), g2.reshape(N))
    return out.reshape(1, N, D)
